# Initial kernel scaffold; baseline (speedup 1.0000x reference)
import jax
import jax.numpy as jnp
from jax.experimental import pallas as pl


def kernel(input, edge_index, edge_weight, temp, scores, bias):
    raise NotImplementedError("write your pallas kernel here")


# speedup vs baseline: 1.6564x; 1.6564x over previous
import jax
import jax.numpy as jnp
from jax import lax
from jax.experimental import pallas as pl
from jax.experimental.pallas import tpu as pltpu
from jax.experimental.pallas import tpu_sc as plsc

N = 10000
E = 160000
D = 256
K = 8
NW = 32
NPT = 320
NPAD = NW * NPT
EC = 128
SCAN = 1600
FLUSH = 1024
STAGE = 1184
ECAP = E + 1024
L = 16
DV = D // L

f32 = jnp.float32
i32 = jnp.int32

_MESH = plsc.VectorSubcoreMesh(core_axis_name="c", subcore_axis_name="s")


def _wid():
    return lax.axis_index("s") * 2 + lax.axis_index("c")


def _hsum(v):
    iot = lax.broadcasted_iota(i32, (L,), 0)
    for sh in (8, 4, 2, 1):
        idx = jnp.bitwise_xor(iot, sh)
        dn = lax.GatherDimensionNumbers(offset_dims=(), collapsed_slice_dims=(0,),
                                        start_index_map=(0,))
        v = v + lax.gather(v, idx[:, None], dn, slice_sizes=(1,),
                           mode=lax.GatherScatterMode.PROMISE_IN_BOUNDS)
    return v


def _prefix_inc(v):
    iot = lax.broadcasted_iota(i32, (L,), 0)
    dn = lax.GatherDimensionNumbers(offset_dims=(), collapsed_slice_dims=(0,),
                                    start_index_map=(0,))
    zero = jnp.zeros((L,), i32)
    for sh in (1, 2, 4, 8):
        idx = jnp.maximum(iot - sh, 0)
        shifted = lax.gather(v, idx[:, None], dn, slice_sizes=(1,),
                             mode=lax.GatherScatterMode.PROMISE_IN_BOUNDS)
        v = v + jnp.where(iot >= sh, shifted, zero)
    return v


def _bf16r(v):
    u = plsc.bitcast(v, i32)
    r = (u + jnp.array(0x7FFF, i32) + ((u >> 16) & jnp.array(1, i32))) & jnp.array(-65536, i32)
    return plsc.bitcast(r, f32)


def _bucket_body(x_hbm, row_hbm, col_hbm, w_hbm, s0_hbm, b0_hbm,
                 ecol_hbm, erow_hbm, ew_hbm, cnt_hbm, hid_hbm,
                 rowb, colb, wb, st_c, st_r, st_w, svec, bvec, blk, cntb):
    wid = _wid()
    lo = wid * NPT
    hi = lo + NPT
    ebase = wid * ECAP

    pltpu.sync_copy(s0_hbm, svec)
    pltpu.sync_copy(b0_hbm, bvec)
    bv = bvec[...]
    sregs = [_bf16r(svec[pl.ds(L * k, L)]) for k in range(DV)]

    def ablk(b, _):
        nb = lo + 32 * b
        pltpu.sync_copy(x_hbm.at[pl.ds(nb, 32)], blk.at[pl.ds(0, 32)])

        def anode(nn, _):
            zacc = jnp.zeros((L,), f32)
            for k in range(DV):
                zacc = zacc + _bf16r(blk[nn, pl.ds(L * k, L)]) * sregs[k]
            zv = _hsum(zacc) + bv
            siv = 1.0 / (1.0 + jnp.exp(-zv))
            for k in range(DV):
                blk[32 + nn, pl.ds(L * k, L)] = siv * blk[nn, pl.ds(L * k, L)]
            return 0

        lax.fori_loop(0, 32, anode, 0)
        pltpu.sync_copy(blk.at[pl.ds(32, 32)], hid_hbm.at[pl.ds(nb, 32)])
        return 0

    lax.fori_loop(0, NPT // 32, ablk, 0)

    def chunk(g, carry):
        base = g * SCAN
        pltpu.sync_copy(row_hbm.at[pl.ds(pl.multiple_of(base, SCAN), SCAN)], rowb)
        pltpu.sync_copy(col_hbm.at[pl.ds(pl.multiple_of(base, SCAN), SCAN)], colb)
        pltpu.sync_copy(w_hbm.at[pl.ds(pl.multiple_of(base, SCAN), SCAN)], wb)

        def grp(j, c2):
            soff, woff = c2
            al = pl.multiple_of(L * j, L)
            rv = rowb[pl.ds(al, L)]
            cv = colb[pl.ds(al, L)]
            wv = wb[pl.ds(al, L)]
            m = (rv >= lo) & (rv < hi)
            mi = jnp.where(m, jnp.array(1, i32), jnp.array(0, i32))
            inc = _prefix_inc(mi)
            trash = lax.broadcasted_iota(i32, (L,), 0) + (STAGE - L)
            idx = jnp.where(m, soff + inc - mi, trash)
            plsc.store_scatter(st_c, [idx], cv)
            plsc.store_scatter(st_r, [idx], rv - lo)
            plsc.store_scatter(st_w, [idx], wv)
            soff = soff + inc[L - 1]
            full = soff >= FLUSH

            @pl.when(full)
            def _flush():
                pltpu.sync_copy(st_c.at[pl.ds(0, FLUSH)],
                                ecol_hbm.at[pl.ds(pl.multiple_of(ebase + woff, FLUSH), FLUSH)])
                pltpu.sync_copy(st_r.at[pl.ds(0, FLUSH)],
                                erow_hbm.at[pl.ds(pl.multiple_of(ebase + woff, FLUSH), FLUSH)])
                pltpu.sync_copy(st_w.at[pl.ds(0, FLUSH)],
                                ew_hbm.at[pl.ds(pl.multiple_of(ebase + woff, FLUSH), FLUSH)])
                tc = st_c[pl.ds(FLUSH, L)]
                tr = st_r[pl.ds(FLUSH, L)]
                tw = st_w[pl.ds(FLUSH, L)]
                st_c[pl.ds(0, L)] = tc
                st_r[pl.ds(0, L)] = tr
                st_w[pl.ds(0, L)] = tw

            soff = jnp.where(full, soff - FLUSH, soff)
            woff = jnp.where(full, woff + FLUSH, woff)
            return (soff, woff)

        return lax.fori_loop(0, SCAN // L, grp, carry)

    soff, woff = lax.fori_loop(0, E // SCAN, chunk, (jnp.array(0, i32), jnp.array(0, i32)))

    dumc = lax.broadcasted_iota(i32, (L,), 0) + wid * L
    zi = jnp.zeros((L,), i32)
    zf = jnp.zeros((L,), f32)
    for q in range(EC // L):
        st_c[pl.ds(soff + L * q, L)] = dumc
        st_r[pl.ds(soff + L * q, L)] = zi
        st_w[pl.ds(soff + L * q, L)] = zf
    total = woff + soff
    ptotal = ((total + (EC - 1)) >> 7) << 7

    def drain(mi, _):
        pltpu.sync_copy(st_c.at[pl.ds(EC * mi, EC)],
                        ecol_hbm.at[pl.ds(pl.multiple_of(ebase + woff + EC * mi, EC), EC)])
        pltpu.sync_copy(st_r.at[pl.ds(EC * mi, EC)],
                        erow_hbm.at[pl.ds(pl.multiple_of(ebase + woff + EC * mi, EC), EC)])
        pltpu.sync_copy(st_w.at[pl.ds(EC * mi, EC)],
                        ew_hbm.at[pl.ds(pl.multiple_of(ebase + woff + EC * mi, EC), EC)])
        return 0

    lax.fori_loop(0, (ptotal - woff) >> 7, drain, 0)
    cntb[...] = jnp.full((L,), ptotal, i32)
    pltpu.sync_copy(cntb, cnt_hbm.at[pl.ds(pl.multiple_of(wid * L, L), L)])


def _hop_body(x_hbm, hid_hbm, ecol_hbm, erow_hbm, ew_hbm, cnt_hbm,
              s_hbm, bv_hbm, tv_hbm,
              xo_hbm, hido_hbm,
              colv, rlocv, wv_v, cntv, rows_v, accs, svec, bvec, tvec, gsem):
    wid = _wid()
    lo = wid * NPT
    ebase = wid * ECAP
    pltpu.sync_copy(cnt_hbm.at[pl.ds(pl.multiple_of(wid * L, L), L)], cntv)
    pltpu.sync_copy(s_hbm, svec)
    pltpu.sync_copy(bv_hbm, bvec)
    pltpu.sync_copy(tv_hbm, tvec)
    nchunks = cntv[...][0] >> 7

    zf = jnp.zeros((L,), f32)

    def zloop(zi, _):
        off = pl.multiple_of(L * zi, L)
        for k in range(DV):
            accs[k][pl.ds(off, L)] = zf
        return 0

    lax.fori_loop(0, NPT, zloop, 0)

    def echunk(g, _):
        eb = g * EC
        pltpu.sync_copy(ecol_hbm.at[pl.ds(pl.multiple_of(ebase + eb, EC), EC)], colv)
        pltpu.sync_copy(erow_hbm.at[pl.ds(pl.multiple_of(ebase + eb, EC), EC)], rlocv)
        pltpu.sync_copy(ew_hbm.at[pl.ds(pl.multiple_of(ebase + eb, EC), EC)], wv_v)
        pltpu.async_copy(x_hbm.at[colv], rows_v, gsem).wait()

        iot = lax.broadcasted_iota(i32, (L,), 0)

        def egrp(eg, _):
            evs = [jnp.full((L,), L * eg + ee, i32) for ee in range(L)]
            bases = [plsc.load_gather(rlocv, [ev]) * L + iot for ev in evs]
            wbs = [plsc.load_gather(wv_v, [ev]) for ev in evs]
            for ee in range(L):
                e = L * eg + ee
                for k in range(DV):
                    plsc.addupdate_scatter(accs[k], [bases[ee]],
                                           wbs[ee] * rows_v[e, pl.ds(L * k, L)])
            return 0

        lax.fori_loop(0, EC // L, egrp, 0)
        return 0

    lax.fori_loop(0, nchunks, echunk, 0)

    tv = tvec[...]
    bv = bvec[...]
    sregs = [_bf16r(svec[pl.ds(L * k, L)]) for k in range(DV)]

    def cblk(b, _):
        nb = lo + 32 * b
        pltpu.sync_copy(x_hbm.at[pl.ds(nb, 32)], rows_v.at[pl.ds(0, 32)])
        pltpu.sync_copy(hid_hbm.at[pl.ds(nb, 32)], rows_v.at[pl.ds(32, 32)])

        def cnode(nn, _):
            abase = (32 * b + nn) * L
            zacc = jnp.zeros((L,), f32)
            for k in range(DV):
                xv = rows_v[nn, pl.ds(L * k, L)] - tv * accs[k][pl.ds(abase, L)]
                rows_v[64 + nn, pl.ds(L * k, L)] = xv
                zacc = zacc + _bf16r(xv) * sregs[k]
            zv = _hsum(zacc) + bv
            siv = 1.0 / (1.0 + jnp.exp(-zv))
            for k in range(DV):
                plsc.addupdate(rows_v.at[32 + nn, pl.ds(L * k, L)],
                               siv * rows_v[64 + nn, pl.ds(L * k, L)])
            return 0

        lax.fori_loop(0, 32, cnode, 0)
        pltpu.sync_copy(rows_v.at[pl.ds(64, 32)], xo_hbm.at[pl.ds(nb, 32)])
        pltpu.sync_copy(rows_v.at[pl.ds(32, 32)], hido_hbm.at[pl.ds(nb, 32)])
        return 0

    lax.fori_loop(0, NPT // 32, cblk, 0)


_bucket_call = pl.kernel(
    _bucket_body,
    out_type=[
        jax.ShapeDtypeStruct((NW * ECAP,), i32),
        jax.ShapeDtypeStruct((NW * ECAP,), i32),
        jax.ShapeDtypeStruct((NW * ECAP,), f32),
        jax.ShapeDtypeStruct((NW * L,), i32),
        jax.ShapeDtypeStruct((NPAD, D), f32),
    ],
    mesh=_MESH,
    compiler_params=pltpu.CompilerParams(needs_layout_passes=False),
    scratch_types=[
        pltpu.VMEM((SCAN,), i32),
        pltpu.VMEM((SCAN,), i32),
        pltpu.VMEM((SCAN,), f32),
        pltpu.VMEM((STAGE,), i32),
        pltpu.VMEM((STAGE,), i32),
        pltpu.VMEM((STAGE,), f32),
        pltpu.VMEM((D,), f32),
        pltpu.VMEM((L,), f32),
        pltpu.VMEM((64, D), f32),
        pltpu.VMEM((L,), i32),
    ],
)

_hop_call = pl.kernel(
    _hop_body,
    out_type=[
        jax.ShapeDtypeStruct((NPAD, D), f32),
        jax.ShapeDtypeStruct((NPAD, D), f32),
    ],
    mesh=_MESH,
    compiler_params=pltpu.CompilerParams(needs_layout_passes=False),
    scratch_types=[
        pltpu.VMEM((EC,), i32),
        pltpu.VMEM((EC,), i32),
        pltpu.VMEM((EC,), f32),
        pltpu.VMEM((L,), i32),
        pltpu.VMEM((EC, D), f32),
        [pltpu.VMEM((NPT * L,), f32) for _ in range(DV)],
        pltpu.VMEM((D,), f32),
        pltpu.VMEM((L,), f32),
        pltpu.VMEM((L,), f32),
        pltpu.SemaphoreType.DMA,
    ],
)


def kernel(input, edge_index, edge_weight, temp, scores, bias):
    x = jnp.concatenate([input.astype(f32), jnp.zeros((NPAD - N, D), f32)], axis=0)
    row = edge_index[0]
    col = edge_index[1]
    w = edge_weight.astype(f32)
    tvals = jnp.tanh(temp.astype(f32))
    s_all = scores[:, :, 0].astype(f32)
    b_all = bias[:, 0].astype(f32)

    ecol, erow, ew, cnt, hid = _bucket_call(
        x, row, col, w, s_all[0], jnp.full((L,), b_all[0], f32))
    for i in range(K):
        x, hid = _hop_call(
            x, hid, ecol, erow, ew, cnt, s_all[i + 1],
            jnp.full((L,), b_all[i + 1], f32),
            jnp.full((L,), tvals[i], f32))
    return hid[:N]
